# baseline (speedup 1.0000x reference)
import functools

import jax
import jax.numpy as jnp
from jax import lax
from jax.experimental import pallas as pl
from jax.experimental.pallas import tpu as pltpu
from jax.experimental.pallas import tpu_sc as plsc

VOCAB = 128000
EMB = 2048
HID = 10240
EPS = 1e-5
BATCH = 4
SEQ = 2048
NTOK = BATCH * SEQ


_SC_NC = 2
_SC_NS = 16
_SC_NW = _SC_NC * _SC_NS
_CH = 16
_NBUF = 3


def _gather_sc(ids2d, table, nrows):
    rows_per_w = nrows // _SC_NW
    nch = rows_per_w // _CH
    mesh = plsc.VectorSubcoreMesh(core_axis_name="c", subcore_axis_name="s")

    @functools.partial(
        pl.kernel,
        mesh=mesh,
        out_type=jax.ShapeDtypeStruct((nrows, EMB), jnp.float32),
        scratch_types=(
            [pltpu.VMEM((nch, _CH), jnp.int32),
             pltpu.VMEM((_NBUF, _CH, EMB), jnp.float32)]
            + [pltpu.SemaphoreType.DMA] * (2 * _NBUF)
        ),
    )
    def k(ids_hbm, table_hbm, out_hbm, idx_v, rows_v, *sems):
        gsem = sems[:_NBUF]
        wsem = sems[_NBUF:]
        wid = lax.axis_index("s") * _SC_NC + lax.axis_index("c")
        base = wid * rows_per_w
        pltpu.sync_copy(ids_hbm.at[pl.ds(wid * nch, nch)], idx_v)
        gcp = [None] * _NBUF
        wcp = [None] * _NBUF

        def start_gather(j):
            b = j % _NBUF
            gcp[b] = pltpu.async_copy(
                table_hbm.at[idx_v.at[j]], rows_v.at[b], gsem[b])

        start_gather(0)
        if nch > 1:
            start_gather(1)
        for j in range(nch):
            b = j % _NBUF
            gcp[b].wait()
            if j + 2 < nch:
                if j >= 1:
                    wcp[(j - 1) % _NBUF].wait()
                start_gather(j + 2)
            wcp[b] = pltpu.async_copy(
                rows_v.at[b], out_hbm.at[pl.ds(base + j * _CH, _CH)], wsem[b])
        for j in range(max(0, nch - _NBUF), nch):
            wcp[j % _NBUF].wait()

    return k(ids2d, table)


LN_BM = 512


def _ln_body(x_ref, g_ref, b_ref, o_ref):
    x = x_ref[...]
    mean = jnp.mean(x, axis=-1, keepdims=True)
    xc = x - mean
    var = jnp.mean(xc * xc, axis=-1, keepdims=True)
    xhat = xc * jax.lax.rsqrt(var + EPS)
    y = xhat * g_ref[...] + b_ref[...]
    o_ref[...] = y.astype(jnp.bfloat16)


def _layernorm_tc(emb, gamma2d, beta2d, nrows):
    return pl.pallas_call(
        _ln_body,
        grid=(nrows // LN_BM,),
        in_specs=[
            pl.BlockSpec((LN_BM, EMB), lambda i: (i, 0)),
            pl.BlockSpec((1, EMB), lambda i: (0, 0)),
            pl.BlockSpec((1, EMB), lambda i: (0, 0)),
        ],
        out_specs=pl.BlockSpec((LN_BM, EMB), lambda i: (i, 0)),
        out_shape=jax.ShapeDtypeStruct((nrows, EMB), jnp.bfloat16),
    )(emb, gamma2d, beta2d)


MM_BM = 2048
MM_BN = 1024


def _mm_body(h_ref, w_ref, b_ref, o_ref):
    acc = jnp.dot(h_ref[...], w_ref[...], preferred_element_type=jnp.float32)
    o_ref[...] = acc + b_ref[...]


def _matmul_tc(h, w_bf16, bias2d, nrows):
    return pl.pallas_call(
        _mm_body,
        grid=(nrows // MM_BM, HID // MM_BN),
        in_specs=[
            pl.BlockSpec((MM_BM, EMB), lambda m, n: (m, 0)),
            pl.BlockSpec((EMB, MM_BN), lambda m, n: (0, n)),
            pl.BlockSpec((1, MM_BN), lambda m, n: (0, n)),
        ],
        out_specs=pl.BlockSpec((MM_BM, MM_BN), lambda m, n: (m, n)),
        out_shape=jax.ShapeDtypeStruct((nrows, HID), jnp.float32),
    )(h, w_bf16, bias2d)


FM_BN = 512
LN_CH = 256


def _fused_body(emb_any, w_ref, b_ref, g_ref, bt_ref, o_ref,
                embv, h_ref, sem0):
    m = pl.program_id(0)
    n = pl.program_id(1)

    @pl.when(n == 0)
    def _prep():
        cp = pltpu.make_async_copy(
            emb_any.at[pl.ds(m * MM_BM, MM_BM), :], embv, sem0)
        cp.start()
        cp.wait()
        for c in range(MM_BM // LN_CH):
            rows = pl.ds(c * LN_CH, LN_CH)
            x = embv[rows, :]
            mean = jnp.mean(x, axis=-1, keepdims=True)
            xc = x - mean
            var = jnp.mean(xc * xc, axis=-1, keepdims=True)
            xhat = xc * jax.lax.rsqrt(var + EPS)
            h_ref[rows, :] = (
                xhat * g_ref[...] + bt_ref[...]).astype(jnp.bfloat16)

    acc = jnp.dot(h_ref[...], w_ref[...], preferred_element_type=jnp.float32)
    o_ref[...] = acc + b_ref[...]


def _fused_mm_ln(emb, w_bf16, bias2d, gamma2d, beta2d):
    return pl.pallas_call(
        _fused_body,
        grid=(NTOK // MM_BM, HID // FM_BN),
        in_specs=[
            pl.BlockSpec(memory_space=pl.ANY),
            pl.BlockSpec((EMB, FM_BN), lambda m, n: (0, n)),
            pl.BlockSpec((1, FM_BN), lambda m, n: (0, n)),
            pl.BlockSpec((1, EMB), lambda m, n: (0, 0)),
            pl.BlockSpec((1, EMB), lambda m, n: (0, 0)),
        ],
        out_specs=pl.BlockSpec((MM_BM, FM_BN), lambda m, n: (m, n)),
        out_shape=jax.ShapeDtypeStruct((NTOK, HID), jnp.float32),
        scratch_shapes=[
            pltpu.VMEM((MM_BM, EMB), jnp.float32),
            pltpu.VMEM((MM_BM, EMB), jnp.bfloat16),
            pltpu.SemaphoreType.DMA,
        ],
        compiler_params=pltpu.CompilerParams(
            dimension_semantics=("arbitrary", "arbitrary")),
    )(emb, w_bf16, bias2d, gamma2d, beta2d)


@jax.jit
def kernel(input_ids, table, ln_gamma, ln_beta, proj_w, proj_b):
    w_bf16 = proj_w.astype(jnp.bfloat16)
    bias2d = proj_b.reshape(1, HID)
    gamma2d = ln_gamma.reshape(1, EMB)
    beta2d = ln_beta.reshape(1, EMB)
    nch = NTOK // _SC_NW // _CH
    ids2d = input_ids.reshape(_SC_NW * nch, _CH).astype(jnp.int32)
    emb = _gather_sc(ids2d, table, NTOK)
    out = _fused_mm_ln(emb, w_bf16, bias2d, gamma2d, beta2d)
    return out.reshape(BATCH, SEQ, HID)


# speedup vs baseline: 1.0446x; 1.0446x over previous
import functools

import jax
import jax.numpy as jnp
from jax import lax
from jax.experimental import pallas as pl
from jax.experimental.pallas import tpu as pltpu
from jax.experimental.pallas import tpu_sc as plsc

VOCAB = 128000
EMB = 2048
HID = 10240
EPS = 1e-5
BATCH = 4
SEQ = 2048
NTOK = BATCH * SEQ


_SC_NC = 2
_SC_NS = 16
_SC_NW = _SC_NC * _SC_NS
_CH = 16
_NBUF = 3


def _gather_sc(ids2d, table, nrows):
    rows_per_w = nrows // _SC_NW
    nch = rows_per_w // _CH
    mesh = plsc.VectorSubcoreMesh(core_axis_name="c", subcore_axis_name="s")

    @functools.partial(
        pl.kernel,
        mesh=mesh,
        out_type=jax.ShapeDtypeStruct((nrows, EMB), jnp.float32),
        scratch_types=(
            [pltpu.VMEM((nch, _CH), jnp.int32),
             pltpu.VMEM((_NBUF, _CH, EMB), jnp.float32)]
            + [pltpu.SemaphoreType.DMA] * (2 * _NBUF)
        ),
    )
    def k(ids_hbm, table_hbm, out_hbm, idx_v, rows_v, *sems):
        gsem = sems[:_NBUF]
        wsem = sems[_NBUF:]
        wid = lax.axis_index("s") * _SC_NC + lax.axis_index("c")
        base = wid * rows_per_w
        pltpu.sync_copy(ids_hbm.at[pl.ds(wid * nch, nch)], idx_v)
        gcp = [None] * _NBUF
        wcp = [None] * _NBUF

        def start_gather(j):
            b = j % _NBUF
            gcp[b] = pltpu.async_copy(
                table_hbm.at[idx_v.at[j]], rows_v.at[b], gsem[b])

        start_gather(0)
        if nch > 1:
            start_gather(1)
        for j in range(nch):
            b = j % _NBUF
            gcp[b].wait()
            if j + 2 < nch:
                if j >= 1:
                    wcp[(j - 1) % _NBUF].wait()
                start_gather(j + 2)
            wcp[b] = pltpu.async_copy(
                rows_v.at[b], out_hbm.at[pl.ds(base + j * _CH, _CH)], wsem[b])
        for j in range(max(0, nch - _NBUF), nch):
            wcp[j % _NBUF].wait()

    return k(ids2d, table)


LN_BM = 1024


def _ln_body(x_ref, g_ref, b_ref, o_ref):
    x = x_ref[...]
    mean = jnp.mean(x, axis=-1, keepdims=True)
    xc = x - mean
    var = jnp.mean(xc * xc, axis=-1, keepdims=True)
    xhat = xc * jax.lax.rsqrt(var + EPS)
    y = xhat * g_ref[...] + b_ref[...]
    o_ref[...] = y.astype(jnp.bfloat16)


def _layernorm_tc(emb, gamma2d, beta2d, nrows):
    return pl.pallas_call(
        _ln_body,
        grid=(nrows // LN_BM,),
        in_specs=[
            pl.BlockSpec((LN_BM, EMB), lambda i: (i, 0)),
            pl.BlockSpec((1, EMB), lambda i: (0, 0)),
            pl.BlockSpec((1, EMB), lambda i: (0, 0)),
        ],
        out_specs=pl.BlockSpec((LN_BM, EMB), lambda i: (i, 0)),
        out_shape=jax.ShapeDtypeStruct((nrows, EMB), jnp.bfloat16),
    )(emb, gamma2d, beta2d)


MM_BM = 2048
MM_BN = 1024


def _mm_body(h_ref, w_ref, b_ref, o_ref):
    acc = jnp.dot(h_ref[...], w_ref[...], preferred_element_type=jnp.float32)
    o_ref[...] = acc + b_ref[...]


def _matmul_tc(h, w_bf16, bias2d, nrows):
    return pl.pallas_call(
        _mm_body,
        grid=(nrows // MM_BM, HID // MM_BN),
        in_specs=[
            pl.BlockSpec((MM_BM, EMB), lambda m, n: (m, 0)),
            pl.BlockSpec((EMB, MM_BN), lambda m, n: (0, n)),
            pl.BlockSpec((1, MM_BN), lambda m, n: (0, n)),
        ],
        out_specs=pl.BlockSpec((MM_BM, MM_BN), lambda m, n: (m, n)),
        out_shape=jax.ShapeDtypeStruct((nrows, HID), jnp.float32),
    )(h, w_bf16, bias2d)


FM_BN = 512
LN_CH = 256


def _fused_body(emb_any, w_ref, b_ref, g_ref, bt_ref, o_ref,
                embv, h_ref, sem0):
    m = pl.program_id(0)
    n = pl.program_id(1)

    @pl.when(n == 0)
    def _prep():
        cp = pltpu.make_async_copy(
            emb_any.at[pl.ds(m * MM_BM, MM_BM), :], embv, sem0)
        cp.start()
        cp.wait()
        for c in range(MM_BM // LN_CH):
            rows = pl.ds(c * LN_CH, LN_CH)
            x = embv[rows, :]
            mean = jnp.mean(x, axis=-1, keepdims=True)
            xc = x - mean
            var = jnp.mean(xc * xc, axis=-1, keepdims=True)
            xhat = xc * jax.lax.rsqrt(var + EPS)
            h_ref[rows, :] = (
                xhat * g_ref[...] + bt_ref[...]).astype(jnp.bfloat16)

    acc = jnp.dot(h_ref[...], w_ref[...], preferred_element_type=jnp.float32)
    o_ref[...] = acc + b_ref[...]


def _fused_mm_ln(emb, w_bf16, bias2d, gamma2d, beta2d):
    return pl.pallas_call(
        _fused_body,
        grid=(NTOK // MM_BM, HID // FM_BN),
        in_specs=[
            pl.BlockSpec(memory_space=pl.ANY),
            pl.BlockSpec((EMB, FM_BN), lambda m, n: (0, n)),
            pl.BlockSpec((1, FM_BN), lambda m, n: (0, n)),
            pl.BlockSpec((1, EMB), lambda m, n: (0, 0)),
            pl.BlockSpec((1, EMB), lambda m, n: (0, 0)),
        ],
        out_specs=pl.BlockSpec((MM_BM, FM_BN), lambda m, n: (m, n)),
        out_shape=jax.ShapeDtypeStruct((NTOK, HID), jnp.float32),
        scratch_shapes=[
            pltpu.VMEM((MM_BM, EMB), jnp.float32),
            pltpu.VMEM((MM_BM, EMB), jnp.bfloat16),
            pltpu.SemaphoreType.DMA,
        ],
        compiler_params=pltpu.CompilerParams(
            dimension_semantics=("arbitrary", "arbitrary")),
    )(emb, w_bf16, bias2d, gamma2d, beta2d)


@jax.jit
def kernel(input_ids, table, ln_gamma, ln_beta, proj_w, proj_b):
    w_bf16 = proj_w.astype(jnp.bfloat16)
    bias2d = proj_b.reshape(1, HID)
    gamma2d = ln_gamma.reshape(1, EMB)
    beta2d = ln_beta.reshape(1, EMB)
    nch = NTOK // _SC_NW // _CH
    ids2d = input_ids.reshape(_SC_NW * nch, _CH).astype(jnp.int32)
    emb = _gather_sc(ids2d, table, NTOK)
    h = _layernorm_tc(emb, gamma2d, beta2d, NTOK)
    out = _matmul_tc(h, w_bf16, bias2d, NTOK)
    return out.reshape(BATCH, SEQ, HID)
